# Initial kernel scaffold; baseline (speedup 1.0000x reference)
import functools

import jax
import jax.numpy as jnp
from jax import lax
from jax.experimental import pallas as pl
from jax.experimental.pallas import tpu as pltpu
from jax.experimental.pallas import tpu_sc as plsc

E = 320000
T = 640000
H = 128
I = 64
NR = 6

NC = 2
NSUB = 16
NW = NC * NSUB
TRI_PER_TILE = T // NW
SCAN_VREGS = TRI_PER_TILE // 16
CHUNK = 24576
NPASS = -(-E // CHUNK)
WIN = 128
DUMP_ROW = CHUNK + 4

BE = 1600
BT = 1600


def _swish(x):
    return x * jax.nn.sigmoid(x)


def _edge_body(rbf0_ref, xkj_ref, wr_ref, wd_ref, o_ref):
    rbf = jnp.dot(rbf0_ref[...], wr_ref[...], preferred_element_type=jnp.float32)
    h = xkj_ref[...] * rbf
    hd = jnp.dot(h, wd_ref[...], preferred_element_type=jnp.float32)
    o_ref[...] = _swish(hd)


def _edge_stage(rbf0, x_kj, Wrbf, Wdown):
    return pl.pallas_call(
        _edge_body,
        grid=(E // BE,),
        in_specs=[
            pl.BlockSpec((BE, NR), lambda i: (i, 0)),
            pl.BlockSpec((BE, H), lambda i: (i, 0)),
            pl.BlockSpec((NR, H), lambda i: (0, 0)),
            pl.BlockSpec((H, I), lambda i: (0, 0)),
        ],
        out_specs=pl.BlockSpec((BE, I), lambda i: (i, 0)),
        out_shape=jax.ShapeDtypeStruct((E, I), jnp.float32),
    )(rbf0, x_kj, Wrbf, Wdown)


def _tri_body(sbf_ref, t_ref, ws_ref, wt_ref, o_ref):
    s = jnp.dot(sbf_ref[...], ws_ref[...], preferred_element_type=jnp.float32)
    tt = jnp.dot(t_ref[...], wt_ref[...], preferred_element_type=jnp.float32)
    o_ref[...] = s * tt


def _tri_stage(sbf, t, Wsbf, Wt):
    return pl.pallas_call(
        _tri_body,
        grid=(T // BT,),
        in_specs=[
            pl.BlockSpec((BT, 18), lambda i: (i, 0)),
            pl.BlockSpec((BT, 54), lambda i: (i, 0)),
            pl.BlockSpec((18, I), lambda i: (0, 0)),
            pl.BlockSpec((54, I), lambda i: (0, 0)),
        ],
        out_specs=pl.BlockSpec((BT, I), lambda i: (i, 0)),
        out_shape=jax.ShapeDtypeStruct((T, I), jnp.float32),
    )(sbf, t, Wsbf, Wt)


def _out_body(p_ref, wu_ref, o_ref):
    acc = p_ref[0] + p_ref[1]
    hu = jnp.dot(acc, wu_ref[...], preferred_element_type=jnp.float32)
    o_ref[...] = _swish(hu)


def _out_stage(partials, Wup):
    return pl.pallas_call(
        _out_body,
        grid=(E // BE,),
        in_specs=[
            pl.BlockSpec((NC, BE, I), lambda i: (0, i, 0)),
            pl.BlockSpec((I, H), lambda i: (0, 0)),
        ],
        out_specs=pl.BlockSpec((BE, H), lambda i: (i, 0)),
        out_shape=jax.ShapeDtypeStruct((E, H), jnp.float32),
    )(partials, Wup)


def _sc_body(h_hbm, m_hbm, kj_hbm, ji_hbm, out_hbm,
             ji_loc, kj_loc, tids, gidx, kjw, dstw, mwin, hwin, acc,
             sem_m, sem_h):
    c = lax.axis_index("c")
    s = lax.axis_index("s")
    wid = c * NSUB + s
    tri_base = wid * TRI_PER_TILE

    pltpu.sync_copy(ji_hbm.at[pl.ds(tri_base, TRI_PER_TILE)], ji_loc)
    pltpu.sync_copy(kj_hbm.at[pl.ds(tri_base, TRI_PER_TILE)], kj_loc)

    iota16 = lax.iota(jnp.int32, 16)
    zeros16 = jnp.zeros((16,), jnp.float32)

    def one_pass(p, carry):
        base = p * CHUNK
        size = jnp.minimum(CHUNK, E - base)
        share = size // NSUB
        row0 = s * share

        def zrow(r, _):
            for j in range(I // 16):
                mwin[r, pl.ds(j * 16, 16)] = zeros16
            return 0
        lax.fori_loop(0, WIN, zrow, 0)
        nz = (share + WIN - 1) // WIN
        def zchunk(i, _):
            pltpu.sync_copy(mwin, acc.at[pl.ds(row0 + i * WIN, WIN)])
            return 0
        lax.fori_loop(0, nz, zchunk, 0)
        plsc.subcore_barrier()

        def scan_body(v, cnt):
            ji = ji_loc[pl.ds(v * 16, 16)]
            msk = (ji >= base) & (ji < base + size)
            plsc.store_compressed(tids.at[pl.ds(cnt, 16)], iota16 + v * 16,
                                  mask=msk)
            return cnt + jnp.sum(msk.astype(jnp.int32))
        cnt = lax.fori_loop(0, SCAN_VREGS, scan_body, 0)

        nwin = (cnt + WIN - 1) // WIN
        def win_body(w, _):
            for j in range(WIN // 16):
                off = w * WIN + j * 16
                tv = tids[pl.ds(off, 16)]
                valid = (off + iota16) < cnt
                tv = jnp.where(valid, tv, 0)
                kj = plsc.load_gather(kj_loc, [tv])
                ji = plsc.load_gather(ji_loc, [tv])
                dst = jnp.where(valid, ji - base, DUMP_ROW)
                gidx[pl.ds(j * 16, 16)] = tv + tri_base
                kjw[pl.ds(j * 16, 16)] = kj
                dstw[0, pl.ds(j * 16, 16)] = dst
            cp_m = pltpu.async_copy(m_hbm.at[gidx], mwin, sem_m)
            cp_h = pltpu.async_copy(h_hbm.at[kjw], hwin, sem_h)
            cp_m.wait()
            cp_h.wait()
            def mulrow(r, _):
                for j in range(I // 16):
                    sl = pl.ds(j * 16, 16)
                    mwin[r, sl] = mwin[r, sl] * hwin[r, sl]
                return 0
            lax.fori_loop(0, WIN, mulrow, 0)
            pltpu.sync_copy(mwin, acc.at[dstw.at[0]], add=True)
            return 0
        lax.fori_loop(0, nwin, win_body, 0)
        plsc.subcore_barrier()

        ncp = share // 32
        def cpchunk(i, _):
            r0 = row0 + i * 32
            pltpu.sync_copy(acc.at[pl.ds(r0, 32)],
                            out_hbm.at[c, pl.ds(base + r0, 32)])
            return 0
        lax.fori_loop(0, ncp, cpchunk, 0)
        plsc.subcore_barrier()
        return carry
    lax.fori_loop(0, NPASS, one_pass, 0)


def _sc_scatter(h, m, idx_kj, idx_ji):
    mesh = plsc.VectorSubcoreMesh(core_axis_name="c", subcore_axis_name="s")
    f = functools.partial(
        pl.kernel,
        out_type=jax.ShapeDtypeStruct((NC, E, I), jnp.float32),
        mesh=mesh,
        scratch_types=[
            pltpu.VMEM((TRI_PER_TILE,), jnp.int32),
            pltpu.VMEM((TRI_PER_TILE,), jnp.int32),
            pltpu.VMEM((TRI_PER_TILE + 16,), jnp.int32),
            pltpu.VMEM((WIN,), jnp.int32),
            pltpu.VMEM((WIN,), jnp.int32),
            pltpu.VMEM((1, WIN), jnp.int32),
            pltpu.VMEM((WIN, I), jnp.float32),
            pltpu.VMEM((WIN, I), jnp.float32),
            pltpu.VMEM_SHARED((CHUNK + WIN + 8, I), jnp.float32),
            pltpu.SemaphoreType.DMA,
            pltpu.SemaphoreType.DMA,
        ],
    )(_sc_body)
    return f(h, m, idx_kj, idx_ji)


def kernel(x1, rbf0, sbf, t, dist_emb_g, x_kj, x_ji,
           Wrbf1, Wrbf2, Wsbf1, Wsbf2, Wt1, Wt2, Wdown, Wup,
           idx_kj, idx_ji):
    Wrbf = Wrbf1 @ Wrbf2
    Wsbf = Wsbf1 @ Wsbf2
    Wt = Wt1 @ Wt2
    h = _edge_stage(rbf0, x_kj, Wrbf, Wdown)
    m = _tri_stage(sbf, t, Wsbf, Wt)
    partials = _sc_scatter(h, m, idx_kj, idx_ji)
    return _out_stage(partials, Wup)


# speedup vs baseline: 1.3065x; 1.3065x over previous
import functools

import jax
import jax.numpy as jnp
from jax import lax
from jax.experimental import pallas as pl
from jax.experimental.pallas import tpu as pltpu
from jax.experimental.pallas import tpu_sc as plsc

E = 320000
T = 640000
H = 128
I = 64
NR = 6

NC = 2
NSUB = 16
NW = NC * NSUB
TRI_PER_TILE = T // NW
SCAN_VREGS = TRI_PER_TILE // 16
CHUNK = 8192
SHIFT = 13
NPASS = -(-E // CHUNK)
WIN = 128
DUMP_ROW = CHUNK + 4

BE = 2560
BT = 2560


def _swish(x):
    return x * jax.nn.sigmoid(x)


def _dot_t(a_t, w):
    return lax.dot_general(a_t, w, (((0,), (0,)), ((), ())),
                           preferred_element_type=jnp.float32)


def _edge_body(rbf0t_ref, xkj_ref, wr_ref, wd_ref, o_ref):
    rbf = _dot_t(rbf0t_ref[...], wr_ref[...])
    h = xkj_ref[...] * rbf
    hd = jnp.dot(h, wd_ref[...], preferred_element_type=jnp.float32)
    o_ref[...] = _swish(hd)


def _edge_stage(rbf0_t, x_kj, Wrbf, Wdown):
    return pl.pallas_call(
        _edge_body,
        grid=(E // BE,),
        in_specs=[
            pl.BlockSpec((NR, BE), lambda i: (0, i)),
            pl.BlockSpec((BE, H), lambda i: (i, 0)),
            pl.BlockSpec((NR, H), lambda i: (0, 0)),
            pl.BlockSpec((H, I), lambda i: (0, 0)),
        ],
        out_specs=pl.BlockSpec((BE, I), lambda i: (i, 0)),
        out_shape=jax.ShapeDtypeStruct((E, I), jnp.float32),
    )(rbf0_t, x_kj, Wrbf, Wdown)


def _tri_body(sbf_ref, t_ref, ws_ref, wt_ref, o_ref):
    s = _dot_t(sbf_ref[...], ws_ref[...])
    tt = _dot_t(t_ref[...], wt_ref[...])
    o_ref[...] = s * tt


def _tri_stage(sbf_t, t_t, Wsbf, Wt):
    return pl.pallas_call(
        _tri_body,
        grid=(T // BT,),
        in_specs=[
            pl.BlockSpec((18, BT), lambda i: (0, i)),
            pl.BlockSpec((54, BT), lambda i: (0, i)),
            pl.BlockSpec((18, I), lambda i: (0, 0)),
            pl.BlockSpec((54, I), lambda i: (0, 0)),
        ],
        out_specs=pl.BlockSpec((BT, I), lambda i: (i, 0)),
        out_shape=jax.ShapeDtypeStruct((T, I), jnp.float32),
    )(sbf_t, t_t, Wsbf, Wt)


def _out_body(p_ref, w2_ref, o_ref):
    acc = p_ref[0] + p_ref[1]
    hu = jnp.dot(acc, w2_ref[...], preferred_element_type=jnp.float32)
    o_ref[...] = _swish(hu)


def _out_stage(partials2, W2):
    return pl.pallas_call(
        _out_body,
        grid=(E // BE,),
        in_specs=[
            pl.BlockSpec((NC, BE // 2, 2 * I), lambda i: (0, i, 0)),
            pl.BlockSpec((2 * I, 2 * H), lambda i: (0, 0)),
        ],
        out_specs=pl.BlockSpec((BE // 2, 2 * H), lambda i: (i, 0)),
        out_shape=jax.ShapeDtypeStruct((E // 2, 2 * H), jnp.float32),
    )(partials2, W2)


def _sc_body(h_hbm, m_hbm, kj_hbm, ji_hbm, out_hbm,
             ji_loc, kj_loc, tids, hist, cursors,
             gidx0, kjw0, dstw0, mwin0, hwin0,
             gidx1, kjw1, dstw1, mwin1, hwin1,
             acc, starts, sem_m0, sem_h0, sem_m1, sem_h1):
    bufs = ((gidx0, kjw0, dstw0, mwin0, hwin0, sem_m0, sem_h0),
            (gidx1, kjw1, dstw1, mwin1, hwin1, sem_m1, sem_h1))
    c = lax.axis_index("c")
    s = lax.axis_index("s")
    wid = c * NSUB + s
    tri_base = pl.multiple_of(wid * TRI_PER_TILE, 32)

    pltpu.sync_copy(ji_hbm.at[pl.ds(tri_base, TRI_PER_TILE)], ji_loc)
    pltpu.sync_copy(kj_hbm.at[pl.ds(tri_base, TRI_PER_TILE)], kj_loc)

    iota16 = lax.iota(jnp.int32, 16)
    zeros16 = jnp.zeros((16,), jnp.float32)
    izeros16 = jnp.zeros((16,), jnp.int32)
    iones16 = jnp.ones((16,), jnp.int32)

    def hzero(b, _):
        hist[pl.ds(b * 16, 16)] = izeros16
        return 0
    lax.fori_loop(0, NPASS, hzero, 0)

    def hscan(v, _):
        ji = ji_loc[pl.ds(v * 16, 16)]
        addr = ((ji >> SHIFT) * 16) + iota16
        plsc.addupdate_scatter(hist, [addr], iones16)
        return 0
    lax.fori_loop(0, SCAN_VREGS, hscan, 0)

    run = jnp.int32(0)
    for b in range(NPASS):
        hv = hist[pl.ds(b * 16, 16)]
        starts[b] = run
        excl = plsc.cumsum(hv) - hv
        cursors[pl.ds(b * 16, 16)] = run + excl
        run = run + jnp.sum(hv)
    starts[NPASS] = run

    def pscan(v, _):
        ji = ji_loc[pl.ds(v * 16, 16)]
        addr = ((ji >> SHIFT) * 16) + iota16
        cur = plsc.load_gather(cursors, [addr])
        plsc.store_scatter(tids, [cur], iota16 + v * 16)
        plsc.addupdate_scatter(cursors, [addr], iones16)
        return 0
    lax.fori_loop(0, SCAN_VREGS, pscan, 0)

    def zwin(r, _):
        for rr in range(4):
            for j in range(I // 16):
                hwin0[r * 4 + rr, pl.ds(j * 16, 16)] = zeros16
        return 0
    lax.fori_loop(0, WIN // 4, zwin, 0)
    ZROW0 = CHUNK // NSUB
    zrow0 = s * ZROW0
    for i in range(ZROW0 // WIN):
        pltpu.sync_copy(hwin0, acc.at[pl.ds(pl.multiple_of(zrow0 + i * WIN, 32), WIN)])
    plsc.subcore_barrier()

    def one_pass(p, carry):
        base = p * CHUNK
        size = jnp.minimum(CHUNK, E - base)
        share = size // NSUB
        row0 = s * share
        lo = starts[p]
        hi = starts[p + 1]
        nwin = (hi - lo + WIN - 1) // WIN

        def build_and_start(w, b):
            gidx, kjw, dstw, mwin, hwin, sem_m, sem_h = bufs[b]
            for j in range(WIN // 16):
                off = lo + w * WIN + j * 16
                tv = tids[pl.ds(off, 16)]
                valid = (off + iota16) < hi
                tv = jnp.where(valid, tv, 0)
                kj = plsc.load_gather(kj_loc, [tv])
                jiv = plsc.load_gather(ji_loc, [tv])
                dst = jnp.where(valid, jiv - base, DUMP_ROW)
                gidx[pl.ds(j * 16, 16)] = tv + tri_base
                kjw[pl.ds(j * 16, 16)] = kj
                dstw[0, pl.ds(j * 16, 16)] = dst
            pltpu.async_copy(m_hbm.at[gidx], mwin, sem_m)
            pltpu.async_copy(h_hbm.at[kjw], hwin, sem_h)

        def consume(b):
            gidx, kjw, dstw, mwin, hwin, sem_m, sem_h = bufs[b]
            pltpu.make_async_copy(m_hbm.at[gidx], mwin, sem_m).wait()
            pltpu.make_async_copy(h_hbm.at[kjw], hwin, sem_h).wait()
            def mulrow(r, _):
                for rr in range(4):
                    row = r * 4 + rr
                    for j in range(I // 16):
                        sl = pl.ds(j * 16, 16)
                        mwin[row, sl] = mwin[row, sl] * hwin[row, sl]
                return 0
            lax.fori_loop(0, WIN // 4, mulrow, 0)
            pltpu.sync_copy(mwin, acc.at[dstw.at[0]], add=True)

        @pl.when(nwin > 0)
        def _():
            build_and_start(0, 0)

        def win_body(w, _):
            nxt = w + 1
            @pl.when(nxt < nwin)
            def _():
                @pl.when(nxt % 2 == 0)
                def _():
                    build_and_start(nxt, 0)
                @pl.when(nxt % 2 == 1)
                def _():
                    build_and_start(nxt, 1)
            @pl.when(w % 2 == 0)
            def _():
                consume(0)
            @pl.when(w % 2 == 1)
            def _():
                consume(1)
            return 0
        lax.fori_loop(0, nwin, win_body, 0)
        plsc.subcore_barrier()

        ncp = share // 32
        def cpchunk(i, _):
            r0 = pl.multiple_of(row0 + i * 32, 32)
            pltpu.sync_copy(acc.at[pl.ds(r0, 32)],
                            out_hbm.at[c, pl.ds(pl.multiple_of(base + r0, 32), 32)])
            return 0
        lax.fori_loop(0, ncp, cpchunk, 0)
        def zwin2(r, _):
            for rr in range(4):
                for j in range(I // 16):
                    hwin0[r * 4 + rr, pl.ds(j * 16, 16)] = zeros16
            return 0
        lax.fori_loop(0, WIN // 4, zwin2, 0)
        for i in range(ZROW0 // WIN):
            pltpu.sync_copy(hwin0,
                            acc.at[pl.ds(pl.multiple_of(zrow0 + i * WIN, 32), WIN)])
        plsc.subcore_barrier()
        return carry
    lax.fori_loop(0, NPASS, one_pass, 0)


def _sc_scatter(h, m, idx_kj, idx_ji):
    mesh = plsc.VectorSubcoreMesh(core_axis_name="c", subcore_axis_name="s")
    winbufs = [
        pltpu.VMEM((WIN,), jnp.int32),
        pltpu.VMEM((WIN,), jnp.int32),
        pltpu.VMEM((1, WIN), jnp.int32),
        pltpu.VMEM((WIN, I), jnp.float32),
        pltpu.VMEM((WIN, I), jnp.float32),
    ]
    f = functools.partial(
        pl.kernel,
        out_type=jax.ShapeDtypeStruct((NC, E, I), jnp.float32),
        mesh=mesh,
        compiler_params=pltpu.CompilerParams(needs_layout_passes=False,
                                             use_tc_tiling_on_sc=False),
        scratch_types=[
            pltpu.VMEM((TRI_PER_TILE,), jnp.int32),
            pltpu.VMEM((TRI_PER_TILE,), jnp.int32),
            pltpu.VMEM((TRI_PER_TILE + WIN,), jnp.int32),
            pltpu.VMEM((NPASS * 16,), jnp.int32),
            pltpu.VMEM((NPASS * 16,), jnp.int32),
        ] + winbufs + winbufs + [
            pltpu.VMEM_SHARED((CHUNK + WIN + 8, I), jnp.float32),
            pltpu.SMEM((NPASS + 8,), jnp.int32),
            pltpu.SemaphoreType.DMA,
            pltpu.SemaphoreType.DMA,
            pltpu.SemaphoreType.DMA,
            pltpu.SemaphoreType.DMA,
        ],
    )(_sc_body)
    return f(h, m, idx_kj, idx_ji)


def kernel(x1, rbf0, sbf, t, dist_emb_g, x_kj, x_ji,
           Wrbf1, Wrbf2, Wsbf1, Wsbf2, Wt1, Wt2, Wdown, Wup,
           idx_kj, idx_ji):
    Wrbf = Wrbf1 @ Wrbf2
    Wsbf = Wsbf1 @ Wsbf2
    Wt = Wt1 @ Wt2
    W2 = jnp.zeros((2 * I, 2 * H), jnp.float32)
    W2 = W2.at[:I, :H].set(Wup).at[I:, H:].set(Wup)
    h = _edge_stage(rbf0.T, x_kj, Wrbf, Wdown)
    m = _tri_stage(sbf.T, t.T, Wsbf, Wt)
    partials = _sc_scatter(h, m, idx_kj, idx_ji)
    out2 = _out_stage(partials.reshape(NC, E // 2, 2 * I), W2)
    return out2.reshape(E, H)
